# baseline (speedup 1.0000x reference)
import functools

import jax
import jax.numpy as jnp
from jax import lax
from jax.experimental import pallas as pl
from jax.experimental.pallas import tpu as pltpu

N = 10000
E = 160000
D_IN = 128
H1 = 64
H2 = 32
EMB = 16


def _stage_a_body(f_ref, w1_ref, ns_ref, o_ref):
    z = jnp.dot(f_ref[...], w1_ref[...], preferred_element_type=jnp.float32)
    o_ref[...] = z * ns_ref[...]


def _stage_a(features, W1, norm_src):
    blk = 1000
    return pl.pallas_call(
        _stage_a_body,
        grid=(N // blk,),
        in_specs=[
            pl.BlockSpec((blk, D_IN), lambda i: (i, 0)),
            pl.BlockSpec((D_IN, H1), lambda i: (0, 0)),
            pl.BlockSpec((blk, 1), lambda i: (i, 0)),
        ],
        out_specs=pl.BlockSpec((blk, H1), lambda i: (i, 0)),
        out_shape=jax.ShapeDtypeStruct((N, H1), jnp.float32),
    )(features, W1, norm_src)


def _stage_b_body(m_ref, nd_ref, b1_ref, w2_ref, wext_ref, ns_ref, o_ref):
    x = jnp.maximum(m_ref[...] * nd_ref[...] + b1_ref[...], 0.0)
    w2e = jnp.dot(w2_ref[...], wext_ref[...], preferred_element_type=jnp.float32)
    o_ref[...] = jnp.dot(x, w2e, preferred_element_type=jnp.float32) * ns_ref[...]


def _stage_b(msg1, norm_dst, b1, W2, Wext, norm_src):
    blk = 1000
    return pl.pallas_call(
        _stage_b_body,
        grid=(N // blk,),
        in_specs=[
            pl.BlockSpec((blk, H1), lambda i: (i, 0)),
            pl.BlockSpec((blk, 1), lambda i: (i, 0)),
            pl.BlockSpec((1, H1), lambda i: (0, 0)),
            pl.BlockSpec((H1, H2), lambda i: (0, 0)),
            pl.BlockSpec((H2, EMB), lambda i: (0, 0)),
            pl.BlockSpec((blk, 1), lambda i: (i, 0)),
        ],
        out_specs=pl.BlockSpec((blk, EMB), lambda i: (i, 0)),
        out_shape=jax.ShapeDtypeStruct((N, EMB), jnp.float32),
    )(msg1, norm_dst, b1, W2, Wext, norm_src)


def _stage_c1_body(m_ref, nd_ref, b2_ref, wext_ref, bext_ref, o_ref):
    b2e = jnp.dot(b2_ref[...], wext_ref[...], preferred_element_type=jnp.float32)
    o_ref[...] = m_ref[...] * nd_ref[...] + b2e + bext_ref[...]


def _stage_c1(msg2, norm_dst, b2, Wext, bext):
    blk = 2000
    return pl.pallas_call(
        _stage_c1_body,
        grid=(N // blk,),
        in_specs=[
            pl.BlockSpec((blk, EMB), lambda i: (i, 0)),
            pl.BlockSpec((blk, 1), lambda i: (i, 0)),
            pl.BlockSpec((1, H2), lambda i: (0, 0)),
            pl.BlockSpec((H2, EMB), lambda i: (0, 0)),
            pl.BlockSpec((1, EMB), lambda i: (0, 0)),
        ],
        out_specs=pl.BlockSpec((blk, EMB), lambda i: (i, 0)),
        out_shape=jax.ShapeDtypeStruct((N, EMB), jnp.float32),
    )(msg2, norm_dst, b2, Wext, bext)


def _stage_c2_body(ei_ref, ej_ref, o_ref):
    g = lax.dot_general(ei_ref[...], ej_ref[...],
                        (((1,), (1,)), ((), ())),
                        preferred_element_type=jnp.float32)
    o_ref[...] = jax.nn.sigmoid(g)


def _stage_c2(emb):
    bi, bj = 512, 1024
    gi = (N + bi - 1) // bi
    gj = (N + bj - 1) // bj
    return pl.pallas_call(
        _stage_c2_body,
        grid=(gi, gj),
        in_specs=[
            pl.BlockSpec((bi, EMB), lambda i, j: (i, 0)),
            pl.BlockSpec((bj, EMB), lambda i, j: (j, 0)),
        ],
        out_specs=pl.BlockSpec((bi, bj), lambda i, j: (i, j)),
        out_shape=jax.ShapeDtypeStruct((N, N), jnp.float32),
    )(emb, emb)


def _norms_body(dp_ref, o_ref):
    deg = jnp.sum(dp_ref[...], axis=0, keepdims=True)
    o_ref[...] = lax.rsqrt(jnp.maximum(deg, 1.0))


def _norms(deg_partials):
    p, m = deg_partials.shape
    return pl.pallas_call(
        _norms_body,
        in_specs=[pl.BlockSpec((p, m), lambda: (0, 0))],
        out_specs=pl.BlockSpec((1, m), lambda: (0, 0)),
        out_shape=jax.ShapeDtypeStruct((1, m), jnp.float32),
    )(deg_partials)


def _degrees(edge_index):
    src, dst = edge_index[0], edge_index[1]
    out_deg = jnp.zeros((N,), jnp.float32).at[src].add(1.0)
    in_deg = jnp.zeros((N,), jnp.float32).at[dst].add(1.0)
    return jnp.concatenate([out_deg, in_deg, jnp.zeros((16,), jnp.float32)])[None]


def _gather_scatter(z, src, dst):
    return jax.ops.segment_sum(jnp.take(z, src, axis=0), dst, num_segments=N)


def kernel(features, edge_index, W1, b1, W2, b2, Wext, bext):
    src, dst = edge_index[0], edge_index[1]

    deg_partials = _degrees(edge_index)
    norms = _norms(deg_partials)[0]
    norm_src = norms[:N].reshape(N, 1)
    norm_dst = norms[N:2 * N].reshape(N, 1)

    z1s = _stage_a(features, W1, norm_src)
    msg1 = _gather_scatter(z1s, src, dst)
    z2s = _stage_b(msg1, norm_dst, b1.reshape(1, H1), W2, Wext, norm_src)
    msg2 = _gather_scatter(z2s, src, dst)
    emb_long = _stage_c1(msg2, norm_dst, b2.reshape(1, H2), Wext,
                         bext.reshape(1, EMB))
    logits = _stage_c2(emb_long)
    return (emb_long, logits)


# speedup vs baseline: 3.8896x; 3.8896x over previous
import functools

import jax
import jax.numpy as jnp
from jax import lax
from jax.experimental import pallas as pl
from jax.experimental.pallas import tpu as pltpu
from jax.experimental.pallas import tpu_sc as plsc

N = 10000
E = 160000
D_IN = 128
H1 = 64
H2 = 32
EMB = 16

NC = 2
NS = 16
NW = NC * NS

N_PAD = N + 112
ROWS_PER_TILE = N_PAD // NS
DEG_M = 2 * N + 16
DEG_EPT = 10240
MP_CHUNK = 128
MP_NCHUNK = 40


def _stage_a_body(f_ref, w1_ref, ns_ref, o_ref):
    z = jnp.dot(f_ref[...], w1_ref[...], preferred_element_type=jnp.float32)
    o_ref[...] = z * ns_ref[...]


def _stage_a(features, W1, norm_src):
    blk = 1000
    return pl.pallas_call(
        _stage_a_body,
        grid=(N // blk,),
        in_specs=[
            pl.BlockSpec((blk, D_IN), lambda i: (i, 0)),
            pl.BlockSpec((D_IN, H1), lambda i: (0, 0)),
            pl.BlockSpec((blk, 1), lambda i: (i, 0)),
        ],
        out_specs=pl.BlockSpec((blk, H1), lambda i: (i, 0)),
        out_shape=jax.ShapeDtypeStruct((N, H1), jnp.float32),
    )(features, W1, norm_src)


def _stage_b_body(m_ref, nd_ref, b1_ref, w2_ref, wext_ref, ns_ref, o_ref):
    m = m_ref[0] + m_ref[1]
    x = jnp.maximum(m * nd_ref[...] + b1_ref[...], 0.0)
    w2e = jnp.dot(w2_ref[...], wext_ref[...], preferred_element_type=jnp.float32)
    o_ref[...] = jnp.dot(x, w2e, preferred_element_type=jnp.float32) * ns_ref[...]


def _stage_b(msg1p, norm_dst, b1, W2, Wext, norm_src):
    blk = 1000
    return pl.pallas_call(
        _stage_b_body,
        grid=(N // blk,),
        in_specs=[
            pl.BlockSpec((NC, blk, H1), lambda i: (0, i, 0)),
            pl.BlockSpec((blk, 1), lambda i: (i, 0)),
            pl.BlockSpec((1, H1), lambda i: (0, 0)),
            pl.BlockSpec((H1, H2), lambda i: (0, 0)),
            pl.BlockSpec((H2, EMB), lambda i: (0, 0)),
            pl.BlockSpec((blk, 1), lambda i: (i, 0)),
        ],
        out_specs=pl.BlockSpec((blk, EMB), lambda i: (i, 0)),
        out_shape=jax.ShapeDtypeStruct((N, EMB), jnp.float32),
    )(msg1p, norm_dst, b1, W2, Wext, norm_src)


def _stage_c1_body(m_ref, nd_ref, b2_ref, wext_ref, bext_ref, o_ref):
    b2e = jnp.dot(b2_ref[...], wext_ref[...], preferred_element_type=jnp.float32)
    o_ref[...] = (m_ref[0] + m_ref[1]) * nd_ref[...] + b2e + bext_ref[...]


def _stage_c1(msg2p, norm_dst, b2, Wext, bext):
    blk = 2000
    return pl.pallas_call(
        _stage_c1_body,
        grid=(N // blk,),
        in_specs=[
            pl.BlockSpec((NC, blk, EMB), lambda i: (0, i, 0)),
            pl.BlockSpec((blk, 1), lambda i: (i, 0)),
            pl.BlockSpec((1, H2), lambda i: (0, 0)),
            pl.BlockSpec((H2, EMB), lambda i: (0, 0)),
            pl.BlockSpec((1, EMB), lambda i: (0, 0)),
        ],
        out_specs=pl.BlockSpec((blk, EMB), lambda i: (i, 0)),
        out_shape=jax.ShapeDtypeStruct((N, EMB), jnp.float32),
    )(msg2p, norm_dst, b2, Wext, bext)


def _stage_c2_body(ei_ref, ej_ref, o_ref):
    g = lax.dot_general(ei_ref[...], ej_ref[...],
                        (((1,), (1,)), ((), ())),
                        preferred_element_type=jnp.float32)
    o_ref[...] = jax.nn.sigmoid(g)


def _stage_c2(emb):
    bi, bj = 512, 1024
    gi = (N + bi - 1) // bi
    gj = (N + bj - 1) // bj
    return pl.pallas_call(
        _stage_c2_body,
        grid=(gi, gj),
        in_specs=[
            pl.BlockSpec((bi, EMB), lambda i, j: (i, 0)),
            pl.BlockSpec((bj, EMB), lambda i, j: (j, 0)),
        ],
        out_specs=pl.BlockSpec((bi, bj), lambda i, j: (i, j)),
        out_shape=jax.ShapeDtypeStruct((N, N), jnp.float32),
    )(emb, emb)


def _norms_body(dp_ref, o_ref):
    deg = jnp.sum(dp_ref[...], axis=0, keepdims=True)
    o_ref[...] = lax.rsqrt(jnp.maximum(deg, 1.0))


def _norms(deg_partials):
    p, m = deg_partials.shape
    return pl.pallas_call(
        _norms_body,
        in_specs=[pl.BlockSpec((p, m), lambda: (0, 0))],
        out_specs=pl.BlockSpec((1, m), lambda: (0, 0)),
        out_shape=jax.ShapeDtypeStruct((1, m), jnp.float32),
    )(deg_partials)


_SC_MESH = plsc.VectorSubcoreMesh(core_axis_name="c", subcore_axis_name="s")
_SC_PARAMS = pltpu.CompilerParams(needs_layout_passes=False,
                                  use_tc_tiling_on_sc=False)


@functools.partial(
    pl.kernel,
    out_type=jax.ShapeDtypeStruct((NW, DEG_M), jnp.float32),
    mesh=_SC_MESH,
    compiler_params=_SC_PARAMS,
    scratch_types=[
        pltpu.VMEM((DEG_EPT,), jnp.int32),
        pltpu.VMEM((DEG_M,), jnp.float32),
    ],
)
def _sc_degrees(idx_hbm, out_hbm, idx_v, acc_v):
    c = lax.axis_index("c")
    s = lax.axis_index("s")
    wid = s * NC + c
    pltpu.sync_copy(idx_hbm.at[wid], idx_v)
    zeros16 = jnp.zeros((16,), jnp.float32)

    def zbody(i, carry):
        acc_v[pl.ds(i * 16, 16)] = zeros16
        return carry

    lax.fori_loop(0, DEG_M // 16, zbody, 0)
    ones16 = jnp.ones((16,), jnp.float32)

    def ebody(i, carry):
        v = idx_v[pl.ds(i * 16, 16)]
        plsc.addupdate_scatter(acc_v, [v], ones16)
        return carry

    lax.fori_loop(0, DEG_EPT // 16, ebody, 0)
    pltpu.sync_copy(acc_v, out_hbm.at[wid])


def _make_sc_mp(W):
    @functools.partial(
        pl.kernel,
        out_type=jax.ShapeDtypeStruct((NC, N_PAD, W), jnp.float32),
        mesh=_SC_MESH,
        compiler_params=_SC_PARAMS,
        scratch_types=[
            pltpu.VMEM((MP_NCHUNK, MP_CHUNK), jnp.int32),
            pltpu.VMEM((MP_NCHUNK, MP_CHUNK), jnp.int32),
            pltpu.VMEM((MP_CHUNK, W), jnp.float32),
            pltpu.VMEM((ROWS_PER_TILE, W), jnp.float32),
            pltpu.VMEM_SHARED((N_PAD, W), jnp.float32),
            pltpu.SemaphoreType.DMA,
        ],
    )
    def mp(z_hbm, src_hbm, dst_hbm, out_hbm, src_v, dst_v, gbuf, rowbuf,
           acc_sh, sem):
        c = lax.axis_index("c")
        s = lax.axis_index("s")
        wid = s * NC + c
        pltpu.sync_copy(src_hbm.at[wid], src_v)
        pltpu.sync_copy(dst_hbm.at[wid], dst_v)
        zeros16 = jnp.zeros((16,), jnp.float32)
        wv = W // 16

        def zbody(i, carry):
            rowbuf[i // wv, pl.ds((i % wv) * 16, 16)] = zeros16
            return carry

        lax.fori_loop(0, ROWS_PER_TILE * wv, zbody, 0)
        pltpu.sync_copy(rowbuf, acc_sh.at[pl.ds(s * ROWS_PER_TILE,
                                                ROWS_PER_TILE)])
        plsc.subcore_barrier()

        def ebody(j, carry):
            pltpu.async_copy(z_hbm.at[src_v.at[j]], gbuf, sem).wait()
            pltpu.sync_copy(gbuf, acc_sh.at[dst_v.at[j]], add=True)
            return carry

        lax.fori_loop(0, MP_NCHUNK, ebody, 0)
        plsc.subcore_barrier()
        pltpu.sync_copy(acc_sh.at[pl.ds(s * ROWS_PER_TILE, ROWS_PER_TILE)],
                        rowbuf)
        pltpu.sync_copy(rowbuf,
                        out_hbm.at[c].at[pl.ds(s * ROWS_PER_TILE,
                                               ROWS_PER_TILE)])

    return mp


_sc_mp64 = _make_sc_mp(H1)
_sc_mp16 = _make_sc_mp(EMB)


def kernel(features, edge_index, W1, b1, W2, b2, Wext, bext):
    src, dst = edge_index[0], edge_index[1]

    pad_e = NW * MP_NCHUNK * MP_CHUNK - E
    src_p = jnp.concatenate(
        [src, jnp.zeros((pad_e,), jnp.int32)]).reshape(NW, MP_NCHUNK, MP_CHUNK)
    dst_p = jnp.concatenate(
        [dst, jnp.full((pad_e,), N, jnp.int32)]).reshape(NW, MP_NCHUNK,
                                                         MP_CHUNK)
    deg_idx = jnp.concatenate(
        [src, dst + N,
         jnp.full((NW * DEG_EPT - 2 * E,), 2 * N, jnp.int32)]).reshape(
             NW, DEG_EPT)

    deg_partials = _sc_degrees(deg_idx)
    norms = _norms(deg_partials)[0]
    norm_src = norms[:N].reshape(N, 1)
    norm_dst = norms[N:2 * N].reshape(N, 1)

    z1s = _stage_a(features, W1, norm_src)
    p1 = _sc_mp64(z1s, src_p, dst_p)
    z2s = _stage_b(p1, norm_dst, b1.reshape(1, H1), W2, Wext, norm_src)
    p2 = _sc_mp16(z2s, src_p, dst_p)
    emb_long = _stage_c1(p2, norm_dst, b2.reshape(1, H2), Wext,
                         bext.reshape(1, EMB))
    logits = _stage_c2(emb_long)
    return (emb_long, logits)


# speedup vs baseline: 4.2385x; 1.0897x over previous
import functools

import jax
import jax.numpy as jnp
from jax import lax
from jax.experimental import pallas as pl
from jax.experimental.pallas import tpu as pltpu
from jax.experimental.pallas import tpu_sc as plsc

N = 10000
E = 160000
D_IN = 128
H1 = 64
H2 = 32
EMB = 16

NC = 2
NS = 16
NW = NC * NS

N_PAD = N + 112
ROWS_PER_TILE = N_PAD // NS
DEG_M = 2 * N + 16
DEG_EPT = 10240
MP_CHUNK = 128
MP_NCHUNK = 40


def _stage_a_body(f_ref, w1_ref, ns_ref, o_ref):
    z = jnp.dot(f_ref[...], w1_ref[...], preferred_element_type=jnp.float32)
    o_ref[...] = z * ns_ref[...]


def _stage_a(features, W1, norm_src):
    blk = 1000
    return pl.pallas_call(
        _stage_a_body,
        grid=(N // blk,),
        in_specs=[
            pl.BlockSpec((blk, D_IN), lambda i: (i, 0)),
            pl.BlockSpec((D_IN, H1), lambda i: (0, 0)),
            pl.BlockSpec((blk, 1), lambda i: (i, 0)),
        ],
        out_specs=pl.BlockSpec((blk, H1), lambda i: (i, 0)),
        out_shape=jax.ShapeDtypeStruct((N, H1), jnp.float32),
    )(features, W1, norm_src)


def _stage_b_body(m_ref, nd_ref, b1_ref, w2_ref, wext_ref, ns_ref, o_ref):
    m = m_ref[0] + m_ref[1]
    x = jnp.maximum(m * nd_ref[...] + b1_ref[...], 0.0)
    w2e = jnp.dot(w2_ref[...], wext_ref[...], preferred_element_type=jnp.float32)
    o_ref[...] = jnp.dot(x, w2e, preferred_element_type=jnp.float32) * ns_ref[...]


def _stage_b(msg1p, norm_dst, b1, W2, Wext, norm_src):
    blk = 1000
    return pl.pallas_call(
        _stage_b_body,
        grid=(N // blk,),
        in_specs=[
            pl.BlockSpec((NC, blk, H1), lambda i: (0, i, 0)),
            pl.BlockSpec((blk, 1), lambda i: (i, 0)),
            pl.BlockSpec((1, H1), lambda i: (0, 0)),
            pl.BlockSpec((H1, H2), lambda i: (0, 0)),
            pl.BlockSpec((H2, EMB), lambda i: (0, 0)),
            pl.BlockSpec((blk, 1), lambda i: (i, 0)),
        ],
        out_specs=pl.BlockSpec((blk, EMB), lambda i: (i, 0)),
        out_shape=jax.ShapeDtypeStruct((N, EMB), jnp.float32),
    )(msg1p, norm_dst, b1, W2, Wext, norm_src)


def _stage_c1_body(m_ref, nd_ref, b2_ref, wext_ref, bext_ref, o_ref):
    b2e = jnp.dot(b2_ref[...], wext_ref[...], preferred_element_type=jnp.float32)
    o_ref[...] = (m_ref[0] + m_ref[1]) * nd_ref[...] + b2e + bext_ref[...]


def _stage_c1(msg2p, norm_dst, b2, Wext, bext):
    blk = 2000
    return pl.pallas_call(
        _stage_c1_body,
        grid=(N // blk,),
        in_specs=[
            pl.BlockSpec((NC, blk, EMB), lambda i: (0, i, 0)),
            pl.BlockSpec((blk, 1), lambda i: (i, 0)),
            pl.BlockSpec((1, H2), lambda i: (0, 0)),
            pl.BlockSpec((H2, EMB), lambda i: (0, 0)),
            pl.BlockSpec((1, EMB), lambda i: (0, 0)),
        ],
        out_specs=pl.BlockSpec((blk, EMB), lambda i: (i, 0)),
        out_shape=jax.ShapeDtypeStruct((N, EMB), jnp.float32),
    )(msg2p, norm_dst, b2, Wext, bext)


def _stage_c2_body(ei_ref, ej_ref, o_ref):
    g = lax.dot_general(ei_ref[...], ej_ref[...],
                        (((1,), (1,)), ((), ())),
                        preferred_element_type=jnp.float32)
    o_ref[...] = jax.nn.sigmoid(g)


def _stage_c2(emb):
    bi, bj = 512, 1024
    gi = (N + bi - 1) // bi
    gj = (N + bj - 1) // bj
    return pl.pallas_call(
        _stage_c2_body,
        grid=(gi, gj),
        in_specs=[
            pl.BlockSpec((bi, EMB), lambda i, j: (i, 0)),
            pl.BlockSpec((bj, EMB), lambda i, j: (j, 0)),
        ],
        out_specs=pl.BlockSpec((bi, bj), lambda i, j: (i, j)),
        out_shape=jax.ShapeDtypeStruct((N, N), jnp.float32),
    )(emb, emb)


def _norms_body(dp_ref, o_ref):
    deg = jnp.sum(dp_ref[...], axis=0, keepdims=True)
    o_ref[...] = lax.rsqrt(jnp.maximum(deg, 1.0))


def _norms(deg_partials):
    p, m = deg_partials.shape
    return pl.pallas_call(
        _norms_body,
        in_specs=[pl.BlockSpec((p, m), lambda: (0, 0))],
        out_specs=pl.BlockSpec((1, m), lambda: (0, 0)),
        out_shape=jax.ShapeDtypeStruct((1, m), jnp.float32),
    )(deg_partials)


_SC_MESH = plsc.VectorSubcoreMesh(core_axis_name="c", subcore_axis_name="s")
_SC_PARAMS = pltpu.CompilerParams(needs_layout_passes=False,
                                  use_tc_tiling_on_sc=False)


@functools.partial(
    pl.kernel,
    out_type=jax.ShapeDtypeStruct((NW, DEG_M), jnp.float32),
    mesh=_SC_MESH,
    compiler_params=_SC_PARAMS,
    scratch_types=[
        pltpu.VMEM((DEG_EPT,), jnp.int32),
        pltpu.VMEM((DEG_M,), jnp.float32),
    ],
)
def _sc_degrees(idx_hbm, out_hbm, idx_v, acc_v):
    c = lax.axis_index("c")
    s = lax.axis_index("s")
    wid = s * NC + c
    pltpu.sync_copy(idx_hbm.at[wid], idx_v)
    zeros16 = jnp.zeros((16,), jnp.float32)

    def zbody(i, carry):
        acc_v[pl.ds(i * 16, 16)] = zeros16
        return carry

    lax.fori_loop(0, DEG_M // 16, zbody, 0)
    ones16 = jnp.ones((16,), jnp.float32)

    def ebody(i, carry):
        v = idx_v[pl.ds(i * 16, 16)]
        plsc.addupdate_scatter(acc_v, [v], ones16)
        return carry

    lax.fori_loop(0, DEG_EPT // 16, ebody, 0)
    pltpu.sync_copy(acc_v, out_hbm.at[wid])


def _make_sc_mp(W):
    nbuf = 8

    @functools.partial(
        pl.kernel,
        out_type=jax.ShapeDtypeStruct((NC, N_PAD, W), jnp.float32),
        mesh=_SC_MESH,
        compiler_params=_SC_PARAMS,
        scratch_types=[
            pltpu.VMEM((MP_NCHUNK, MP_CHUNK), jnp.int32),
            pltpu.VMEM((MP_NCHUNK, MP_CHUNK), jnp.int32),
            pltpu.VMEM((nbuf, MP_CHUNK, W), jnp.float32),
            pltpu.VMEM_SHARED((N_PAD, W), jnp.float32),
        ] + [pltpu.SemaphoreType.DMA] * (2 * nbuf),
    )
    def mp(z_hbm, src_hbm, dst_hbm, out_hbm, src_v, dst_v, gbuf,
           acc_sh, *sems):
        gsem = sems[:nbuf]
        ssem = sems[nbuf:]
        c = lax.axis_index("c")
        s = lax.axis_index("s")
        wid = s * NC + c
        pltpu.sync_copy(src_hbm.at[wid], src_v)
        pltpu.sync_copy(dst_hbm.at[wid], dst_v)
        zeros16 = jnp.zeros((16,), jnp.float32)
        wv = W // 16

        def zbody(i, carry):
            gbuf[0, i // wv, pl.ds((i % wv) * 16, 16)] = zeros16
            return carry

        lax.fori_loop(0, MP_CHUNK * wv, zbody, 0)
        row0 = s * ROWS_PER_TILE
        chunks = []
        off = 0
        while off < ROWS_PER_TILE:
            sz = min(MP_CHUNK, ROWS_PER_TILE - off)
            chunks.append((off, sz))
            off += sz
        for off, sz in chunks:
            pltpu.sync_copy(gbuf.at[0].at[pl.ds(0, sz)],
                            acc_sh.at[pl.ds(row0 + off, sz)])
        plsc.subcore_barrier()

        def gather(cid, b):
            return pltpu.async_copy(z_hbm.at[src_v.at[cid]], gbuf.at[b],
                                    gsem[b])

        def scatter(cid, b):
            return pltpu.async_copy(gbuf.at[b], acc_sh.at[dst_v.at[cid]],
                                    ssem[b], add=True)

        def ebody(j, carry):
            base = j * nbuf
            gds = [gather(base + b, b) for b in range(nbuf)]
            sds = []
            for b in range(nbuf):
                gds[b].wait()
                sds.append(scatter(base + b, b))
            for b in range(nbuf):
                sds[b].wait()
            return carry

        lax.fori_loop(0, MP_NCHUNK // nbuf, ebody, 0)
        plsc.subcore_barrier()
        for off, sz in chunks:
            pltpu.sync_copy(acc_sh.at[pl.ds(row0 + off, sz)],
                            gbuf.at[0].at[pl.ds(0, sz)])
            pltpu.sync_copy(gbuf.at[0].at[pl.ds(0, sz)],
                            out_hbm.at[c].at[pl.ds(row0 + off, sz)])

    return mp


_sc_mp64 = _make_sc_mp(H1)
_sc_mp16 = _make_sc_mp(EMB)


def kernel(features, edge_index, W1, b1, W2, b2, Wext, bext):
    src, dst = edge_index[0], edge_index[1]

    pad_e = NW * MP_NCHUNK * MP_CHUNK - E
    src_p = jnp.concatenate(
        [src, jnp.zeros((pad_e,), jnp.int32)]).reshape(NW, MP_NCHUNK, MP_CHUNK)
    dst_p = jnp.concatenate(
        [dst, jnp.full((pad_e,), N, jnp.int32)]).reshape(NW, MP_NCHUNK,
                                                         MP_CHUNK)
    deg_idx = jnp.concatenate(
        [src, dst + N,
         jnp.full((NW * DEG_EPT - 2 * E,), 2 * N, jnp.int32)]).reshape(
             NW, DEG_EPT)

    deg_partials = _sc_degrees(deg_idx)
    norms = _norms(deg_partials)[0]
    norm_src = norms[:N].reshape(N, 1)
    norm_dst = norms[N:2 * N].reshape(N, 1)

    z1s = _stage_a(features, W1, norm_src)
    p1 = _sc_mp64(z1s, src_p, dst_p)
    z2s = _stage_b(p1, norm_dst, b1.reshape(1, H1), W2, Wext, norm_src)
    p2 = _sc_mp16(z2s, src_p, dst_p)
    emb_long = _stage_c1(p2, norm_dst, b2.reshape(1, H2), Wext,
                         bext.reshape(1, EMB))
    logits = _stage_c2(emb_long)
    return (emb_long, logits)


# speedup vs baseline: 4.3083x; 1.0165x over previous
import functools

import jax
import jax.numpy as jnp
from jax import lax
from jax.experimental import pallas as pl
from jax.experimental.pallas import tpu as pltpu
from jax.experimental.pallas import tpu_sc as plsc

N = 10000
E = 160000
D_IN = 128
H1 = 64
H2 = 32
EMB = 16

NC = 2
NS = 16
NW = NC * NS

N_PAD = N + 112
ROWS_PER_TILE = N_PAD // NS
DEG_M = 2 * N + 16
DEG_EPT = 10240
MP_CHUNK = 128
MP_NCHUNK = 40


def _stage_a_body(f_ref, w1_ref, ns_ref, o_ref):
    z = jnp.dot(f_ref[...], w1_ref[...], preferred_element_type=jnp.float32)
    o_ref[...] = z * ns_ref[...]


def _stage_a(features, W1, norm_src):
    blk = 1000
    return pl.pallas_call(
        _stage_a_body,
        grid=(N // blk,),
        in_specs=[
            pl.BlockSpec((blk, D_IN), lambda i: (i, 0)),
            pl.BlockSpec((D_IN, H1), lambda i: (0, 0)),
            pl.BlockSpec((blk, 1), lambda i: (i, 0)),
        ],
        out_specs=pl.BlockSpec((blk, H1), lambda i: (i, 0)),
        out_shape=jax.ShapeDtypeStruct((N, H1), jnp.float32),
    )(features, W1, norm_src)


def _stage_b_body(m_ref, nd_ref, b1_ref, w2_ref, wext_ref, ns_ref, o_ref):
    m = m_ref[0] + m_ref[1]
    x = jnp.maximum(m * nd_ref[...] + b1_ref[...], 0.0)
    w2e = jnp.dot(w2_ref[...], wext_ref[...], preferred_element_type=jnp.float32)
    o_ref[...] = jnp.dot(x, w2e, preferred_element_type=jnp.float32) * ns_ref[...]


def _stage_b(msg1p, norm_dst, b1, W2, Wext, norm_src):
    blk = 1000
    return pl.pallas_call(
        _stage_b_body,
        grid=(N // blk,),
        in_specs=[
            pl.BlockSpec((NC, blk, H1), lambda i: (0, i, 0)),
            pl.BlockSpec((blk, 1), lambda i: (i, 0)),
            pl.BlockSpec((1, H1), lambda i: (0, 0)),
            pl.BlockSpec((H1, H2), lambda i: (0, 0)),
            pl.BlockSpec((H2, EMB), lambda i: (0, 0)),
            pl.BlockSpec((blk, 1), lambda i: (i, 0)),
        ],
        out_specs=pl.BlockSpec((blk, EMB), lambda i: (i, 0)),
        out_shape=jax.ShapeDtypeStruct((N, EMB), jnp.float32),
    )(msg1p, norm_dst, b1, W2, Wext, norm_src)


def _stage_c1_body(m_ref, nd_ref, b2_ref, wext_ref, bext_ref, o_ref):
    b2e = jnp.dot(b2_ref[...], wext_ref[...], preferred_element_type=jnp.float32)
    o_ref[...] = (m_ref[0] + m_ref[1]) * nd_ref[...] + b2e + bext_ref[...]


def _stage_c1(msg2p, norm_dst, b2, Wext, bext):
    blk = 2000
    return pl.pallas_call(
        _stage_c1_body,
        grid=(N // blk,),
        in_specs=[
            pl.BlockSpec((NC, blk, EMB), lambda i: (0, i, 0)),
            pl.BlockSpec((blk, 1), lambda i: (i, 0)),
            pl.BlockSpec((1, H2), lambda i: (0, 0)),
            pl.BlockSpec((H2, EMB), lambda i: (0, 0)),
            pl.BlockSpec((1, EMB), lambda i: (0, 0)),
        ],
        out_specs=pl.BlockSpec((blk, EMB), lambda i: (i, 0)),
        out_shape=jax.ShapeDtypeStruct((N, EMB), jnp.float32),
    )(msg2p, norm_dst, b2, Wext, bext)


def _stage_c2_body(ei_ref, ej_ref, o_ref):
    g = lax.dot_general(ei_ref[...], ej_ref[...],
                        (((1,), (1,)), ((), ())),
                        preferred_element_type=jnp.float32)
    o_ref[...] = jax.nn.sigmoid(g)


def _stage_c2(emb):
    bi, bj = 512, 1024
    gi = (N + bi - 1) // bi
    gj = (N + bj - 1) // bj
    return pl.pallas_call(
        _stage_c2_body,
        grid=(gi, gj),
        in_specs=[
            pl.BlockSpec((bi, EMB), lambda i, j: (i, 0)),
            pl.BlockSpec((bj, EMB), lambda i, j: (j, 0)),
        ],
        out_specs=pl.BlockSpec((bi, bj), lambda i, j: (i, j)),
        out_shape=jax.ShapeDtypeStruct((N, N), jnp.float32),
    )(emb, emb)


def _norms_body(dp_ref, o_ref):
    deg = jnp.sum(dp_ref[...], axis=0, keepdims=True)
    o_ref[...] = lax.rsqrt(jnp.maximum(deg, 1.0))


def _norms(deg_partials):
    p, m = deg_partials.shape
    return pl.pallas_call(
        _norms_body,
        in_specs=[pl.BlockSpec((p, m), lambda: (0, 0))],
        out_specs=pl.BlockSpec((1, m), lambda: (0, 0)),
        out_shape=jax.ShapeDtypeStruct((1, m), jnp.float32),
    )(deg_partials)


_SC_MESH = plsc.VectorSubcoreMesh(core_axis_name="c", subcore_axis_name="s")
_SC_PARAMS = pltpu.CompilerParams(needs_layout_passes=False,
                                  use_tc_tiling_on_sc=False)


@functools.partial(
    pl.kernel,
    out_type=jax.ShapeDtypeStruct((NW, DEG_M), jnp.float32),
    mesh=_SC_MESH,
    compiler_params=_SC_PARAMS,
    scratch_types=[
        pltpu.VMEM((DEG_EPT,), jnp.int32),
        pltpu.VMEM((DEG_M,), jnp.float32),
    ],
)
def _sc_degrees(idx_hbm, out_hbm, idx_v, acc_v):
    c = lax.axis_index("c")
    s = lax.axis_index("s")
    wid = s * NC + c
    pltpu.sync_copy(idx_hbm.at[wid], idx_v)
    zeros16 = jnp.zeros((16,), jnp.float32)

    def zbody(i, carry):
        acc_v[pl.ds(i * 16, 16)] = zeros16
        return carry

    lax.fori_loop(0, DEG_M // 16, zbody, 0)
    ones16 = jnp.ones((16,), jnp.float32)

    def ebody(i, carry):
        v = idx_v[pl.ds(i * 16, 16)]
        plsc.addupdate_scatter(acc_v, [v], ones16)
        return carry

    lax.fori_loop(0, DEG_EPT // 16, ebody, 0)
    pltpu.sync_copy(acc_v, out_hbm.at[wid])


def _make_sc_mp(W):
    nbuf = 8

    @functools.partial(
        pl.kernel,
        out_type=jax.ShapeDtypeStruct((NC, N_PAD, W), jnp.float32),
        mesh=_SC_MESH,
        compiler_params=_SC_PARAMS,
        scratch_types=[
            pltpu.VMEM((MP_NCHUNK, MP_CHUNK), jnp.int32),
            pltpu.VMEM((MP_NCHUNK, MP_CHUNK), jnp.int32),
            pltpu.VMEM((nbuf, MP_CHUNK, W), jnp.float32),
            pltpu.VMEM_SHARED((N_PAD, W), jnp.float32),
        ] + [pltpu.SemaphoreType.DMA] * (2 * nbuf),
    )
    def mp(z_hbm, src_hbm, dst_hbm, out_hbm, src_v, dst_v, gbuf,
           acc_sh, *sems):
        gsem = sems[:nbuf]
        ssem = sems[nbuf:]
        c = lax.axis_index("c")
        s = lax.axis_index("s")
        wid = s * NC + c
        pltpu.sync_copy(src_hbm.at[wid], src_v)
        pltpu.sync_copy(dst_hbm.at[wid], dst_v)
        zeros16 = jnp.zeros((16,), jnp.float32)
        wv = W // 16

        def zbody(i, carry):
            gbuf[0, i // wv, pl.ds((i % wv) * 16, 16)] = zeros16
            return carry

        lax.fori_loop(0, MP_CHUNK * wv, zbody, 0)
        row0 = s * ROWS_PER_TILE
        chunks = []
        off = 0
        while off < ROWS_PER_TILE:
            sz = min(MP_CHUNK, ROWS_PER_TILE - off)
            chunks.append((off, sz))
            off += sz
        for off, sz in chunks:
            pltpu.sync_copy(gbuf.at[0].at[pl.ds(0, sz)],
                            acc_sh.at[pl.ds(row0 + off, sz)])
        plsc.subcore_barrier()

        def gather(cid, b):
            return pltpu.async_copy(z_hbm.at[src_v.at[cid]], gbuf.at[b],
                                    gsem[b])

        def scatter(cid, b):
            return pltpu.async_copy(gbuf.at[b], acc_sh.at[dst_v.at[cid]],
                                    ssem[b], add=True)

        look = 4
        gdesc = {c: gather(c, c % nbuf) for c in range(look)}
        sdesc = {}
        for cid in range(MP_NCHUNK):
            gdesc[cid].wait()
            sdesc[cid] = scatter(cid, cid % nbuf)
            nxt = cid + look
            if nxt < MP_NCHUNK:
                if nxt - nbuf in sdesc:
                    sdesc[nxt - nbuf].wait()
                gdesc[nxt] = gather(nxt, nxt % nbuf)
        for cid in range(MP_NCHUNK - nbuf, MP_NCHUNK):
            sdesc[cid].wait()
        plsc.subcore_barrier()
        for off, sz in chunks:
            pltpu.sync_copy(acc_sh.at[pl.ds(row0 + off, sz)],
                            gbuf.at[0].at[pl.ds(0, sz)])
            pltpu.sync_copy(gbuf.at[0].at[pl.ds(0, sz)],
                            out_hbm.at[c].at[pl.ds(row0 + off, sz)])

    return mp


_sc_mp64 = _make_sc_mp(H1)
_sc_mp16 = _make_sc_mp(EMB)


def kernel(features, edge_index, W1, b1, W2, b2, Wext, bext):
    src, dst = edge_index[0], edge_index[1]

    pad_e = NW * MP_NCHUNK * MP_CHUNK - E
    src_p = jnp.concatenate(
        [src, jnp.zeros((pad_e,), jnp.int32)]).reshape(NW, MP_NCHUNK, MP_CHUNK)
    dst_p = jnp.concatenate(
        [dst, jnp.full((pad_e,), N, jnp.int32)]).reshape(NW, MP_NCHUNK,
                                                         MP_CHUNK)
    deg_idx = jnp.concatenate(
        [src, dst + N,
         jnp.full((NW * DEG_EPT - 2 * E,), 2 * N, jnp.int32)]).reshape(
             NW, DEG_EPT)

    deg_partials = _sc_degrees(deg_idx)
    norms = _norms(deg_partials)[0]
    norm_src = norms[:N].reshape(N, 1)
    norm_dst = norms[N:2 * N].reshape(N, 1)

    z1s = _stage_a(features, W1, norm_src)
    p1 = _sc_mp64(z1s, src_p, dst_p)
    z2s = _stage_b(p1, norm_dst, b1.reshape(1, H1), W2, Wext, norm_src)
    p2 = _sc_mp16(z2s, src_p, dst_p)
    emb_long = _stage_c1(p2, norm_dst, b2.reshape(1, H2), Wext,
                         bext.reshape(1, EMB))
    logits = _stage_c2(emb_long)
    return (emb_long, logits)


# speedup vs baseline: 4.7598x; 1.1048x over previous
import functools

import jax
import jax.numpy as jnp
from jax import lax
from jax.experimental import pallas as pl
from jax.experimental.pallas import tpu as pltpu
from jax.experimental.pallas import tpu_sc as plsc

N = 10000
E = 160000
D_IN = 128
H1 = 64
H2 = 32
EMB = 16

NC = 2
NS = 16
NW = NC * NS

N_PAD = N + 112
ROWS_PER_TILE = N_PAD // NS
DEG_M = 2 * N + 16
DEG_EPT = 10240
MP_CHUNK = 128
MP_NCHUNK = 40


def _stage_a_body(f_ref, w1_ref, o_ref):
    o_ref[...] = jnp.dot(f_ref[...], w1_ref[...],
                         preferred_element_type=jnp.float32)


def _stage_a(features, W1):
    blk = 1000
    return pl.pallas_call(
        _stage_a_body,
        grid=(N // blk,),
        in_specs=[
            pl.BlockSpec((blk, D_IN), lambda i: (i, 0)),
            pl.BlockSpec((D_IN, H1), lambda i: (0, 0)),
        ],
        out_specs=pl.BlockSpec((blk, H1), lambda i: (i, 0)),
        out_shape=jax.ShapeDtypeStruct((N, H1), jnp.float32),
    )(features, W1)


def _scale_body(z_ref, ns_ref, o_ref):
    o_ref[...] = (z_ref[...] * ns_ref[...]).astype(jnp.bfloat16)


def _scale(z1, norm_src):
    blk = 1000
    return pl.pallas_call(
        _scale_body,
        grid=(N // blk,),
        in_specs=[
            pl.BlockSpec((blk, H1), lambda i: (i, 0)),
            pl.BlockSpec((blk, 1), lambda i: (i, 0)),
        ],
        out_specs=pl.BlockSpec((blk, H1), lambda i: (i, 0)),
        out_shape=jax.ShapeDtypeStruct((N, H1), jnp.bfloat16),
    )(z1, norm_src)


def _stage_b_body(m_ref, nd_ref, b1_ref, w2_ref, wext_ref, ns_ref, o_ref):
    m = m_ref[0].astype(jnp.float32) + m_ref[1].astype(jnp.float32)
    x = jnp.maximum(m * nd_ref[...] + b1_ref[...], 0.0)
    w2e = jnp.dot(w2_ref[...], wext_ref[...], preferred_element_type=jnp.float32)
    o_ref[...] = jnp.dot(x, w2e, preferred_element_type=jnp.float32) * ns_ref[...]


def _stage_b(msg1p, norm_dst, b1, W2, Wext, norm_src):
    blk = 1000
    return pl.pallas_call(
        _stage_b_body,
        grid=(N // blk,),
        in_specs=[
            pl.BlockSpec((NC, blk, H1), lambda i: (0, i, 0)),
            pl.BlockSpec((blk, 1), lambda i: (i, 0)),
            pl.BlockSpec((1, H1), lambda i: (0, 0)),
            pl.BlockSpec((H1, H2), lambda i: (0, 0)),
            pl.BlockSpec((H2, EMB), lambda i: (0, 0)),
            pl.BlockSpec((blk, 1), lambda i: (i, 0)),
        ],
        out_specs=pl.BlockSpec((blk, EMB), lambda i: (i, 0)),
        out_shape=jax.ShapeDtypeStruct((N, EMB), jnp.float32),
    )(msg1p, norm_dst, b1, W2, Wext, norm_src)


def _stage_c1_body(m_ref, nd_ref, b2_ref, wext_ref, bext_ref, o_ref):
    b2e = jnp.dot(b2_ref[...], wext_ref[...], preferred_element_type=jnp.float32)
    o_ref[...] = (m_ref[0] + m_ref[1]) * nd_ref[...] + b2e + bext_ref[...]


def _stage_c1(msg2p, norm_dst, b2, Wext, bext):
    blk = 2000
    return pl.pallas_call(
        _stage_c1_body,
        grid=(N // blk,),
        in_specs=[
            pl.BlockSpec((NC, blk, EMB), lambda i: (0, i, 0)),
            pl.BlockSpec((blk, 1), lambda i: (i, 0)),
            pl.BlockSpec((1, H2), lambda i: (0, 0)),
            pl.BlockSpec((H2, EMB), lambda i: (0, 0)),
            pl.BlockSpec((1, EMB), lambda i: (0, 0)),
        ],
        out_specs=pl.BlockSpec((blk, EMB), lambda i: (i, 0)),
        out_shape=jax.ShapeDtypeStruct((N, EMB), jnp.float32),
    )(msg2p, norm_dst, b2, Wext, bext)


def _stage_c2_body(ei_ref, ej_ref, o_ref):
    g = lax.dot_general(ei_ref[...], ej_ref[...],
                        (((1,), (1,)), ((), ())),
                        preferred_element_type=jnp.float32)
    o_ref[...] = jax.nn.sigmoid(g)


def _stage_c2(emb):
    bi, bj = 512, 1024
    gi = (N + bi - 1) // bi
    gj = (N + bj - 1) // bj
    return pl.pallas_call(
        _stage_c2_body,
        grid=(gi, gj),
        in_specs=[
            pl.BlockSpec((bi, EMB), lambda i, j: (i, 0)),
            pl.BlockSpec((bj, EMB), lambda i, j: (j, 0)),
        ],
        out_specs=pl.BlockSpec((bi, bj), lambda i, j: (i, j)),
        out_shape=jax.ShapeDtypeStruct((N, N), jnp.float32),
    )(emb, emb)


def _norms_body(dp_ref, o_ref):
    deg = jnp.sum(dp_ref[...], axis=0, keepdims=True)
    o_ref[...] = lax.rsqrt(jnp.maximum(deg, 1.0))


def _norms(deg_partials):
    p, m = deg_partials.shape
    return pl.pallas_call(
        _norms_body,
        in_specs=[pl.BlockSpec((p, m), lambda: (0, 0))],
        out_specs=pl.BlockSpec((1, m), lambda: (0, 0)),
        out_shape=jax.ShapeDtypeStruct((1, m), jnp.float32),
    )(deg_partials)


_SC_MESH = plsc.VectorSubcoreMesh(core_axis_name="c", subcore_axis_name="s")
_SC_PARAMS = pltpu.CompilerParams(needs_layout_passes=False,
                                  use_tc_tiling_on_sc=False)


@functools.partial(
    pl.kernel,
    out_type=jax.ShapeDtypeStruct((NW, DEG_M), jnp.float32),
    mesh=_SC_MESH,
    compiler_params=_SC_PARAMS,
    scratch_types=[
        pltpu.VMEM((DEG_EPT,), jnp.int32),
        pltpu.VMEM((DEG_M,), jnp.float32),
    ],
)
def _sc_degrees(idx_hbm, out_hbm, idx_v, acc_v):
    c = lax.axis_index("c")
    s = lax.axis_index("s")
    wid = s * NC + c
    pltpu.sync_copy(idx_hbm.at[wid], idx_v)
    zeros16 = jnp.zeros((16,), jnp.float32)

    def zbody(i, carry):
        acc_v[pl.ds(i * 16, 16)] = zeros16
        return carry

    lax.fori_loop(0, DEG_M // 16, zbody, 0)
    ones16 = jnp.ones((16,), jnp.float32)

    def ebody(i, carry):
        v = idx_v[pl.ds(i * 16, 16)]
        plsc.addupdate_scatter(acc_v, [v], ones16)
        return carry

    lax.fori_loop(0, DEG_EPT // 16, ebody, 0)
    pltpu.sync_copy(acc_v, out_hbm.at[wid])


def _make_sc_mp(W, dtype):
    nbuf = 8
    lanes = 16 if dtype == jnp.float32 else 32

    @functools.partial(
        pl.kernel,
        out_type=jax.ShapeDtypeStruct((NC, N_PAD, W), dtype),
        mesh=_SC_MESH,
        compiler_params=_SC_PARAMS,
        scratch_types=[
            pltpu.VMEM((MP_NCHUNK, MP_CHUNK), jnp.int32),
            pltpu.VMEM((MP_NCHUNK, MP_CHUNK), jnp.int32),
            pltpu.VMEM((nbuf, MP_CHUNK, W), dtype),
            pltpu.VMEM_SHARED((N_PAD, W), dtype),
        ] + [pltpu.SemaphoreType.DMA] * (2 * nbuf),
    )
    def mp(z_hbm, src_hbm, dst_hbm, out_hbm, src_v, dst_v, gbuf,
           acc_sh, *sems):
        gsem = sems[:nbuf]
        ssem = sems[nbuf:]
        c = lax.axis_index("c")
        s = lax.axis_index("s")
        wid = s * NC + c
        pltpu.sync_copy(src_hbm.at[wid], src_v)
        pltpu.sync_copy(dst_hbm.at[wid], dst_v)
        zvec = jnp.zeros((lanes,), dtype)
        wv = W // lanes

        def zbody(i, carry):
            gbuf[0, i // wv, pl.ds((i % wv) * lanes, lanes)] = zvec
            return carry

        lax.fori_loop(0, MP_CHUNK * wv, zbody, 0)
        row0 = s * ROWS_PER_TILE
        chunks = []
        off = 0
        while off < ROWS_PER_TILE:
            sz = min(MP_CHUNK, ROWS_PER_TILE - off)
            chunks.append((off, sz))
            off += sz
        for off, sz in chunks:
            pltpu.sync_copy(gbuf.at[0].at[pl.ds(0, sz)],
                            acc_sh.at[pl.ds(row0 + off, sz)])
        plsc.subcore_barrier()

        def gather(cid, b):
            return pltpu.async_copy(z_hbm.at[src_v.at[cid]], gbuf.at[b],
                                    gsem[b])

        def scatter(cid, b):
            return pltpu.async_copy(gbuf.at[b], acc_sh.at[dst_v.at[cid]],
                                    ssem[b], add=True)

        look = 4
        gdesc = {c: gather(c, c % nbuf) for c in range(look)}
        sdesc = {}
        for cid in range(MP_NCHUNK):
            gdesc[cid].wait()
            sdesc[cid] = scatter(cid, cid % nbuf)
            nxt = cid + look
            if nxt < MP_NCHUNK:
                if nxt - nbuf in sdesc:
                    sdesc[nxt - nbuf].wait()
                gdesc[nxt] = gather(nxt, nxt % nbuf)
        for cid in range(MP_NCHUNK - nbuf, MP_NCHUNK):
            sdesc[cid].wait()
        plsc.subcore_barrier()
        for off, sz in chunks:
            pltpu.sync_copy(acc_sh.at[pl.ds(row0 + off, sz)],
                            gbuf.at[0].at[pl.ds(0, sz)])
            pltpu.sync_copy(gbuf.at[0].at[pl.ds(0, sz)],
                            out_hbm.at[c].at[pl.ds(row0 + off, sz)])

    return mp


_sc_mp64 = _make_sc_mp(H1, jnp.bfloat16)
_sc_mp16 = _make_sc_mp(EMB, jnp.float32)


def kernel(features, edge_index, W1, b1, W2, b2, Wext, bext):
    src, dst = edge_index[0], edge_index[1]

    pad_e = NW * MP_NCHUNK * MP_CHUNK - E
    src_p = jnp.concatenate(
        [src, jnp.zeros((pad_e,), jnp.int32)]).reshape(NW, MP_NCHUNK, MP_CHUNK)
    dst_p = jnp.concatenate(
        [dst, jnp.full((pad_e,), N, jnp.int32)]).reshape(NW, MP_NCHUNK,
                                                         MP_CHUNK)
    deg_idx = jnp.concatenate(
        [src, dst + N,
         jnp.full((NW * DEG_EPT - 2 * E,), 2 * N, jnp.int32)]).reshape(
             NW, DEG_EPT)

    deg_partials = _sc_degrees(deg_idx)
    norms = _norms(deg_partials)[0]
    norm_src = norms[:N].reshape(N, 1)
    norm_dst = norms[N:2 * N].reshape(N, 1)

    z1 = _stage_a(features, W1)
    z1s = _scale(z1, norm_src)
    p1 = _sc_mp64(z1s, src_p, dst_p)
    z2s = _stage_b(p1, norm_dst, b1.reshape(1, H1), W2, Wext, norm_src)
    p2 = _sc_mp16(z2s, src_p, dst_p)
    emb_long = _stage_c1(p2, norm_dst, b2.reshape(1, H2), Wext,
                         bext.reshape(1, EMB))
    logits = _stage_c2(emb_long)
    return (emb_long, logits)
